# baseline (speedup 1.0000x reference)
import functools

import jax
import jax.numpy as jnp
from jax import lax
from jax.experimental import pallas as pl

_POOLING_FACTOR = 0.25


def _rmax(a):
    return jnp.max(jnp.max(a, axis=0, keepdims=True), axis=1, keepdims=True)


def _rmin(a):
    return jnp.min(jnp.min(a, axis=0, keepdims=True), axis=1, keepdims=True)


def _rsum(a):
    return jnp.sum(jnp.sum(a, axis=0, keepdims=True), axis=1, keepdims=True)


def _fps_kernel(xr, yr, zr, out_ref, *, b, rows, n, m):
    iot = (lax.broadcasted_iota(jnp.int32, (rows, 128), 0) * 128
           + lax.broadcasted_iota(jnp.int32, (rows, 128), 1)
           ).astype(jnp.float32)
    big = jnp.float32(n)
    X = [xr[i] for i in range(b)]
    Y = [yr[i] for i in range(b)]
    Z = [zr[i] for i in range(b)]

    def body(i, carry):
        new_carry = []
        for bi in range(b):
            d, cx, cy, cz = carry[bi]
            out_ref[bi, pl.ds(i, 1), :] = jnp.concatenate([cx, cy, cz], axis=1)
            dd = (X[bi] - cx) ** 2 + ((Y[bi] - cy) ** 2 + (Z[bi] - cz) ** 2)
            d = jnp.minimum(d, dd)
            mx = _rmax(d)
            idx = _rmin(jnp.where(d == mx, iot, big))
            oh = iot == idx
            cx = _rsum(jnp.where(oh, X[bi], 0.0))
            cy = _rsum(jnp.where(oh, Y[bi], 0.0))
            cz = _rsum(jnp.where(oh, Z[bi], 0.0))
            new_carry.append((d, cx, cy, cz))
        return tuple(new_carry)

    init = tuple(
        (jnp.full((rows, 128), jnp.inf, dtype=jnp.float32),
         X[bi][0:1, 0:1], Y[bi][0:1, 0:1], Z[bi][0:1, 0:1])
        for bi in range(b))
    lax.fori_loop(0, m, body, init, unroll=False)


def kernel(x, offset):
    b = offset.shape[0]
    n = x.shape[0] // b
    m = int(n * _POOLING_FACTOR)
    rows = n // 128
    coords = x[:, :3]
    xr = coords[:, 0].reshape(b, rows, 128)
    yr = coords[:, 1].reshape(b, rows, 128)
    zr = coords[:, 2].reshape(b, rows, 128)
    out = pl.pallas_call(
        functools.partial(_fps_kernel, b=b, rows=rows, n=n, m=m),
        out_shape=jax.ShapeDtypeStruct((b, m, 3), jnp.float32),
    )(xr, yr, zr)
    return out.reshape(b * m, 3)


# speedup vs baseline: 1.4314x; 1.4314x over previous
import functools

import jax
import jax.numpy as jnp
from jax import lax
from jax.experimental import pallas as pl
from jax.experimental.pallas import tpu as pltpu

_POOLING_FACTOR = 0.25


def _lexmax(ta, tb):
    da, ia, xa, ya, za = ta
    db, ib, xb, yb, zb = tb
    keep = (da > db) | ((da == db) & (ia < ib))
    sel = lambda p, q: jnp.where(keep, p, q)
    return (sel(da, db), sel(ia, ib), sel(xa, xb), sel(ya, yb), sel(za, zb))


def _keepfirst(tbest, tnew):
    keep = tbest[0] >= tnew[0]
    sel = lambda p, q: jnp.where(keep, p, q)
    return tuple(sel(p, q) for p, q in zip(tbest, tnew))


def _sroll(a, s):
    return jnp.concatenate([a[s:], a[:s]], axis=0)


def _fps_kernel(xr, yr, zr, ox_ref, oy_ref, oz_ref, d_ref, *, b, rows, n, m):
    nblk = rows // 8
    riota = lax.broadcasted_iota(
        jnp.int32, (rows, 128), 0).astype(jnp.float32)
    lane_i = lax.broadcasted_iota(jnp.int32, (8, 128), 1)
    d_ref[...] = jnp.full((b, rows, 128), jnp.inf, dtype=jnp.float32)

    def body(i, carry):
        for bi in range(b):
            cx, cy, cz = carry[bi]
            ox_ref[pl.ds(i, 1), bi:bi + 1] = cx
            oy_ref[pl.ds(i, 1), bi:bi + 1] = cy
            oz_ref[pl.ds(i, 1), bi:bi + 1] = cz
        bests = [None] * b
        for k in range(nblk):
            r = slice(8 * k, 8 * (k + 1))
            for bi in range(b):
                cx, cy, cz = carry[bi]
                xb = xr[bi, r]
                yb = yr[bi, r]
                zb = zr[bi, r]
                dd = (xb - cx) ** 2 + ((yb - cy) ** 2 + (zb - cz) ** 2)
                dblk = jnp.minimum(d_ref[bi, r], dd)
                d_ref[bi, r] = dblk
                t = (dblk, riota[r], xb, yb, zb)
                bests[bi] = t if k == 0 else _keepfirst(bests[bi], t)
        for s in (4, 2, 1):
            for bi in range(b):
                t = bests[bi]
                bests[bi] = _lexmax(t, tuple(_sroll(a, s) for a in t))
        ncarry = []
        for bi in range(b):
            d8, _, x8, y8, z8 = bests[bi]
            li = jnp.argmax(d8, axis=1, keepdims=True)
            oh = lane_i == li
            cx = jnp.sum(jnp.where(oh, x8, 0.0), axis=1, keepdims=True)
            cy = jnp.sum(jnp.where(oh, y8, 0.0), axis=1, keepdims=True)
            cz = jnp.sum(jnp.where(oh, z8, 0.0), axis=1, keepdims=True)
            ncarry.append((cx[0:1], cy[0:1], cz[0:1]))
        return tuple(ncarry)

    init = tuple(
        (xr[bi, 0:1, 0:1], yr[bi, 0:1, 0:1], zr[bi, 0:1, 0:1])
        for bi in range(b))
    lax.fori_loop(0, m, body, init, unroll=False)


def kernel(x, offset):
    b = offset.shape[0]
    n = x.shape[0] // b
    m = int(n * _POOLING_FACTOR)
    rows = n // 128
    coords = x[:, :3]
    xr = coords[:, 0].reshape(b, 128, rows).swapaxes(1, 2)
    yr = coords[:, 1].reshape(b, 128, rows).swapaxes(1, 2)
    zr = coords[:, 2].reshape(b, 128, rows).swapaxes(1, 2)
    ox, oy, oz = pl.pallas_call(
        functools.partial(_fps_kernel, b=b, rows=rows, n=n, m=m),
        out_shape=tuple(
            jax.ShapeDtypeStruct((m, b), jnp.float32) for _ in range(3)),
        scratch_shapes=[pltpu.VMEM((b, rows, 128), jnp.float32)],
    )(xr, yr, zr)
    out = jnp.stack([ox, oy, oz], axis=-1)
    return out.transpose(1, 0, 2).reshape(b * m, 3)


# speedup vs baseline: 1.6412x; 1.1465x over previous
import functools

import jax
import jax.numpy as jnp
from jax import lax
from jax.experimental import pallas as pl
from jax.experimental.pallas import tpu as pltpu

_POOLING_FACTOR = 0.25


def _lexmax(ta, tb):
    da, ia, xa, ya, za = ta
    db, ib, xb, yb, zb = tb
    keep = (da > db) | ((da == db) & (ia < ib))
    sel = lambda p, q: jnp.where(keep, p, q)
    return (sel(da, db), sel(ia, ib), sel(xa, xb), sel(ya, yb), sel(za, zb))


def _keepfirst(tbest, tnew):
    keep = tbest[0] >= tnew[0]
    sel = lambda p, q: jnp.where(keep, p, q)
    return tuple(sel(p, q) for p, q in zip(tbest, tnew))


def _sroll(a, s):
    return jnp.concatenate([a[s:], a[:s]], axis=0)


def _fps_kernel(xr, yr, zr, ox_ref, oy_ref, oz_ref, d_ref, *, b, rows, n, m):
    nblk = rows // 8
    riota = lax.broadcasted_iota(
        jnp.int32, (rows, 128), 0).astype(jnp.float32)
    lane_i = lax.broadcasted_iota(jnp.int32, (8, 128), 1)
    d_ref[...] = jnp.full((b, rows, 128), jnp.inf, dtype=jnp.float32)

    def body(i, carry):
        for bi in range(b):
            cx, cy, cz = carry[bi]
            ox_ref[pl.ds(i, 1), bi:bi + 1] = cx
            oy_ref[pl.ds(i, 1), bi:bi + 1] = cy
            oz_ref[pl.ds(i, 1), bi:bi + 1] = cz
        bests = [None] * b
        for k in range(nblk):
            r = slice(8 * k, 8 * (k + 1))
            for bi in range(b):
                cx, cy, cz = carry[bi]
                xb = xr[bi, r]
                yb = yr[bi, r]
                zb = zr[bi, r]
                dd = ((xb - cx) ** 2 + (zb - cz) ** 2) + (yb - cy) ** 2
                dblk = jnp.minimum(d_ref[bi, r], dd)
                d_ref[bi, r] = dblk
                t = (dblk, riota[r], xb, yb, zb)
                bests[bi] = t if k == 0 else _keepfirst(bests[bi], t)
        for s in (4, 2, 1):
            for bi in range(b):
                t = bests[bi]
                bests[bi] = _lexmax(t, tuple(_sroll(a, s) for a in t))
        ncarry = []
        for bi in range(b):
            d8, _, x8, y8, z8 = bests[bi]
            li = jnp.argmax(d8, axis=1, keepdims=True)
            oh = lane_i == li
            cx = jnp.sum(jnp.where(oh, x8, 0.0), axis=1, keepdims=True)
            cy = jnp.sum(jnp.where(oh, y8, 0.0), axis=1, keepdims=True)
            cz = jnp.sum(jnp.where(oh, z8, 0.0), axis=1, keepdims=True)
            ncarry.append((cx[0:1], cy[0:1], cz[0:1]))
        return tuple(ncarry)

    m0 = (lane_i == 127) & (lax.broadcasted_iota(jnp.int32, (8, 128), 0) == 0)
    init = tuple(
        tuple(jnp.sum(jnp.where(m0, p[bi, 0:8], 0.0),
                      axis=1, keepdims=True)[0:1]
              for p in (xr, yr, zr))
        for bi in range(b))
    init = body(0, init)
    lax.fori_loop(1, m, body, init, unroll=False)


def kernel(x, offset):
    b = offset.shape[0]
    n = x.shape[0] // b
    m = int(n * _POOLING_FACTOR)
    rows = n // 128
    coords = x[:, :3]
    xr = coords[:, 0].reshape(b, 128, rows)[:, ::-1, :].swapaxes(1, 2)
    yr = coords[:, 1].reshape(b, 128, rows)[:, ::-1, :].swapaxes(1, 2)
    zr = coords[:, 2].reshape(b, 128, rows)[:, ::-1, :].swapaxes(1, 2)
    ox, oy, oz = pl.pallas_call(
        functools.partial(_fps_kernel, b=b, rows=rows, n=n, m=m),
        out_shape=tuple(
            jax.ShapeDtypeStruct((m, b), jnp.float32) for _ in range(3)),
        scratch_shapes=[pltpu.VMEM((b, rows, 128), jnp.float32)],
    )(xr, yr, zr)
    out = jnp.stack([ox, oy, oz], axis=-1)
    return out.transpose(1, 0, 2).reshape(b * m, 3)


# speedup vs baseline: 1.6418x; 1.0004x over previous
import functools

import jax
import jax.numpy as jnp
from jax import lax
from jax.experimental import pallas as pl
from jax.experimental.pallas import tpu as pltpu

_POOLING_FACTOR = 0.25


def _lexmax(ta, tb):
    da, ia, xa, ya, za = ta
    db, ib, xb, yb, zb = tb
    keep = (da > db) | ((da == db) & (ia < ib))
    sel = lambda p, q: jnp.where(keep, p, q)
    return (sel(da, db), sel(ia, ib), sel(xa, xb), sel(ya, yb), sel(za, zb))


def _keepfirst(tbest, tnew):
    keep = tbest[0] >= tnew[0]
    sel = lambda p, q: jnp.where(keep, p, q)
    return tuple(sel(p, q) for p, q in zip(tbest, tnew))


def _sroll(a, s):
    return jnp.concatenate([a[s:], a[:s]], axis=0)


def _fps_kernel(xr, yr, zr, ox_ref, oy_ref, oz_ref, d_ref, *, b, rows, m):
    nblk = rows // 8
    riota = lax.broadcasted_iota(
        jnp.int32, (rows, 128), 0).astype(jnp.float32)
    lane_i = lax.broadcasted_iota(jnp.int32, (8, 128), 1)
    d_ref[...] = jnp.full((b, rows, 128), jnp.inf, dtype=jnp.float32)

    def body(i, carry):
        for bi in range(b):
            cx, cy, cz = carry[bi]
            ox_ref[pl.ds(i, 1), bi:bi + 1] = cx
            oy_ref[pl.ds(i, 1), bi:bi + 1] = cy
            oz_ref[pl.ds(i, 1), bi:bi + 1] = cz
        bests = [None] * b
        for k in range(nblk):
            r = slice(8 * k, 8 * (k + 1))
            for bi in range(b):
                cx, cy, cz = carry[bi]
                xb = xr[bi, r]
                yb = yr[bi, r]
                zb = zr[bi, r]
                dd = ((xb - cx) ** 2 + (zb - cz) ** 2) + (yb - cy) ** 2
                dblk = jnp.minimum(d_ref[bi, r], dd)
                d_ref[bi, r] = dblk
                t = (dblk, riota[r], xb, yb, zb)
                bests[bi] = t if k == 0 else _keepfirst(bests[bi], t)
        for s in (4, 2, 1):
            for bi in range(b):
                t = bests[bi]
                bests[bi] = _lexmax(t, tuple(_sroll(a, s) for a in t))
        ncarry = []
        for bi in range(b):
            d8, _, x8, y8, z8 = bests[bi]
            li = jnp.argmax(d8, axis=1, keepdims=True)
            oh = lane_i == li
            cx = jnp.sum(jnp.where(oh, x8, 0.0), axis=1, keepdims=True)
            cy = jnp.sum(jnp.where(oh, y8, 0.0), axis=1, keepdims=True)
            cz = jnp.sum(jnp.where(oh, z8, 0.0), axis=1, keepdims=True)
            ncarry.append((cx[0:1], cy[0:1], cz[0:1]))
        return tuple(ncarry)

    m0 = (lane_i == 127) & (lax.broadcasted_iota(jnp.int32, (8, 128), 0) == 0)
    init = tuple(
        tuple(jnp.sum(jnp.where(m0, p[bi, 0:8], 0.0),
                      axis=1, keepdims=True)[0:1]
              for p in (xr, yr, zr))
        for bi in range(b))
    init = body(0, init)
    lax.fori_loop(1, m, body, init, unroll=False)


def kernel(x, offset):
    b = offset.shape[0]
    n = x.shape[0] // b
    m = int(n * _POOLING_FACTOR)
    rows = n // 128
    coords = x[:, :3]
    xr = coords[:, 0].reshape(b, 128, rows)[:, ::-1, :].swapaxes(1, 2)
    yr = coords[:, 1].reshape(b, 128, rows)[:, ::-1, :].swapaxes(1, 2)
    zr = coords[:, 2].reshape(b, 128, rows)[:, ::-1, :].swapaxes(1, 2)
    ox, oy, oz = pl.pallas_call(
        functools.partial(_fps_kernel, b=b, rows=rows, m=m),
        out_shape=tuple(
            jax.ShapeDtypeStruct((m, b), jnp.float32) for _ in range(3)),
        scratch_shapes=[pltpu.VMEM((b, rows, 128), jnp.float32)],
    )(xr, yr, zr)
    out = jnp.stack([ox, oy, oz], axis=-1)
    return out.transpose(1, 0, 2).reshape(b * m, 3)
